# Initial kernel scaffold; baseline (speedup 1.0000x reference)
import jax
import jax.numpy as jnp
from jax.experimental import pallas as pl


def kernel(x, weight, bias):
    raise NotImplementedError("write your pallas kernel here")


# speedup vs baseline: 1.4055x; 1.4055x over previous
import functools

import jax
import jax.numpy as jnp
from jax.experimental import pallas as pl

NUM_EXPERTS = 64
TOP_K = 8
TOKEN_BLOCK = 2048


def _gate_kernel(x_ref, w_ref, b_ref, weights_ref, indices_ref):
    x = x_ref[...]
    w = w_ref[...]
    b = b_ref[...]

    scores = jax.lax.dot_general(
        x, w, (((1,), (1,)), ((), ())),
        preferred_element_type=jnp.float32)

    m = jnp.max(scores, axis=-1, keepdims=True)
    e = jnp.exp(scores - m)
    probs = e / jnp.sum(e, axis=-1, keepdims=True)

    routing = probs + b

    B = routing.shape[0]
    iota = jax.lax.broadcasted_iota(jnp.int32, (B, NUM_EXPERTS), 1)

    w_cols = []
    i_cols = []
    for _ in range(TOP_K):
        mx = jnp.max(routing, axis=-1, keepdims=True)
        is_max = routing == mx
        idx = jnp.min(jnp.where(is_max, iota, NUM_EXPERTS), axis=-1,
                      keepdims=True)
        sel = iota == idx
        w_cols.append(jnp.sum(jnp.where(sel, probs, 0.0), axis=-1,
                              keepdims=True))
        i_cols.append(idx)
        routing = jnp.where(sel, -jnp.inf, routing)

    weights_ref[...] = jnp.concatenate(w_cols, axis=-1)
    indices_ref[...] = jnp.concatenate(i_cols, axis=-1)


@jax.jit
def kernel(x, weight, bias):
    n_tokens, hidden = x.shape
    grid = (n_tokens // TOKEN_BLOCK,)
    bias2d = bias.reshape(1, NUM_EXPERTS)

    weights, indices = pl.pallas_call(
        _gate_kernel,
        grid=grid,
        in_specs=[
            pl.BlockSpec((TOKEN_BLOCK, hidden), lambda i: (i, 0)),
            pl.BlockSpec((NUM_EXPERTS, hidden), lambda i: (0, 0)),
            pl.BlockSpec((1, NUM_EXPERTS), lambda i: (0, 0)),
        ],
        out_specs=[
            pl.BlockSpec((TOKEN_BLOCK, TOP_K), lambda i: (i, 0)),
            pl.BlockSpec((TOKEN_BLOCK, TOP_K), lambda i: (i, 0)),
        ],
        out_shape=[
            jax.ShapeDtypeStruct((n_tokens, TOP_K), jnp.float32),
            jax.ShapeDtypeStruct((n_tokens, TOP_K), jnp.int32),
        ],
    )(x, weight, bias2d)

    return weights.astype(x.dtype), indices


# speedup vs baseline: 3.5350x; 2.5151x over previous
import jax
import jax.numpy as jnp
from jax.experimental import pallas as pl

NUM_EXPERTS = 64
TOP_K = 8
TOKEN_BLOCK = 2048


def _gate_kernel(x_ref, w_ref, b_ref, weights_ref, indices_ref):
    x = x_ref[...]
    w = w_ref[...]
    b = b_ref[...]

    scores = jax.lax.dot_general(
        w, x, (((1,), (1,)), ((), ())),
        preferred_element_type=jnp.float32)

    m = jnp.max(scores, axis=0, keepdims=True)
    e = jnp.exp(scores - m)
    probs = e * (1.0 / jnp.sum(e, axis=0, keepdims=True))

    routing = probs + b

    B = routing.shape[1]
    iota = jax.lax.broadcasted_iota(jnp.int32, (NUM_EXPERTS, B), 0)
    packed = jax.lax.bitwise_or(
        jax.lax.bitwise_and(
            jax.lax.bitcast_convert_type(routing, jnp.int32) + 32,
            jnp.int32(~63)),
        (NUM_EXPERTS - 1) - iota)
    keys = jax.lax.bitcast_convert_type(packed, jnp.float32)

    w_rows = []
    i_rows = []
    for _ in range(TOP_K):
        mx = jnp.max(keys, axis=0, keepdims=True)
        idx = (NUM_EXPERTS - 1) - jax.lax.bitwise_and(
            jax.lax.bitcast_convert_type(mx, jnp.int32), 63)
        sel = iota == idx
        w_rows.append(jnp.max(jnp.where(sel, probs, -1.0), axis=0,
                              keepdims=True))
        i_rows.append(idx)
        keys = jnp.where(sel, -jnp.inf, keys)

    weights_ref[...] = jnp.concatenate(w_rows, axis=0).T
    indices_ref[...] = jnp.concatenate(i_rows, axis=0).T


@jax.jit
def kernel(x, weight, bias):
    n_tokens, hidden = x.shape
    grid = (n_tokens // TOKEN_BLOCK,)
    bias2d = bias.reshape(NUM_EXPERTS, 1)

    weights, indices = pl.pallas_call(
        _gate_kernel,
        grid=grid,
        in_specs=[
            pl.BlockSpec((TOKEN_BLOCK, hidden), lambda i: (i, 0)),
            pl.BlockSpec((NUM_EXPERTS, hidden), lambda i: (0, 0)),
            pl.BlockSpec((NUM_EXPERTS, 1), lambda i: (0, 0)),
        ],
        out_specs=[
            pl.BlockSpec((TOKEN_BLOCK, TOP_K), lambda i: (i, 0)),
            pl.BlockSpec((TOKEN_BLOCK, TOP_K), lambda i: (i, 0)),
        ],
        out_shape=[
            jax.ShapeDtypeStruct((n_tokens, TOP_K), jnp.float32),
            jax.ShapeDtypeStruct((n_tokens, TOP_K), jnp.int32),
        ],
    )(x, weight, bias2d)

    return weights.astype(x.dtype), indices


# speedup vs baseline: 3.6584x; 1.0349x over previous
import jax
import jax.numpy as jnp
from jax.experimental import pallas as pl

NUM_EXPERTS = 64
TOP_K = 8
TOKEN_BLOCK = 4096


def _gate_kernel(x_ref, w_ref, b_ref, weights_ref, indices_ref):
    x = x_ref[...]
    w = w_ref[...]
    b = b_ref[...]

    scores = jax.lax.dot_general(
        w, x, (((1,), (1,)), ((), ())),
        preferred_element_type=jnp.float32)

    m = jnp.max(scores, axis=0, keepdims=True)
    e = jnp.exp(scores - m)
    probs = e * (1.0 / jnp.sum(e, axis=0, keepdims=True))

    routing = probs + b

    B = routing.shape[1]
    iota = jax.lax.broadcasted_iota(jnp.int32, (NUM_EXPERTS, B), 0)
    packed = jax.lax.bitwise_or(
        jax.lax.bitwise_and(
            jax.lax.bitcast_convert_type(routing, jnp.int32) + 32,
            jnp.int32(~63)),
        (NUM_EXPERTS - 1) - iota)
    keys = jax.lax.bitcast_convert_type(packed, jnp.float32)

    w_rows = []
    i_rows = []
    for _ in range(TOP_K):
        mx = jnp.max(keys, axis=0, keepdims=True)
        idx = (NUM_EXPERTS - 1) - jax.lax.bitwise_and(
            jax.lax.bitcast_convert_type(mx, jnp.int32), 63)
        sel = iota == idx
        w_rows.append(jnp.max(jnp.where(sel, probs, -1.0), axis=0,
                              keepdims=True))
        i_rows.append(idx)
        keys = jnp.where(sel, -jnp.inf, keys)

    weights_ref[...] = jnp.concatenate(w_rows, axis=0).T
    indices_ref[...] = jnp.concatenate(i_rows, axis=0).T


@jax.jit
def kernel(x, weight, bias):
    n_tokens, hidden = x.shape
    grid = (n_tokens // TOKEN_BLOCK,)
    bias2d = bias.reshape(NUM_EXPERTS, 1)

    weights, indices = pl.pallas_call(
        _gate_kernel,
        grid=grid,
        in_specs=[
            pl.BlockSpec((TOKEN_BLOCK, hidden), lambda i: (i, 0)),
            pl.BlockSpec((NUM_EXPERTS, hidden), lambda i: (0, 0)),
            pl.BlockSpec((NUM_EXPERTS, 1), lambda i: (0, 0)),
        ],
        out_specs=[
            pl.BlockSpec((TOKEN_BLOCK, TOP_K), lambda i: (i, 0)),
            pl.BlockSpec((TOKEN_BLOCK, TOP_K), lambda i: (i, 0)),
        ],
        out_shape=[
            jax.ShapeDtypeStruct((n_tokens, TOP_K), jnp.float32),
            jax.ShapeDtypeStruct((n_tokens, TOP_K), jnp.int32),
        ],
    )(x, weight, bias2d)

    return weights.astype(x.dtype), indices
